# baseline (speedup 1.0000x reference)
import functools

import jax
import jax.numpy as jnp
from jax import lax
from jax.experimental import pallas as pl
from jax.experimental.pallas import tpu as pltpu
from jax.experimental.pallas import tpu_sc as plsc

BATCH = 16384
EMBED_DIM = 64
NUM_USERS = 1000000
NUM_MOVIES = 100000
NUM_GENRES = 1000
NUM_CORES = 2
NUM_SUBCORES = 16
NUM_WORKERS = NUM_CORES * NUM_SUBCORES
CW = 40704
NVEC = BATCH // 16
TABLES = ((NUM_USERS,), (NUM_MOVIES,), (NUM_GENRES,))


def _gather3(uid, mid, gid, utT, mtT, gtT, utail, mtail, gtail):
    mesh = plsc.VectorSubcoreMesh(core_axis_name="c", subcore_axis_name="s")
    out = jax.ShapeDtypeStruct((EMBED_DIM, BATCH), jnp.float32)

    @functools.partial(
        pl.kernel,
        mesh=mesh,
        out_type=(out, out, out),
        compiler_params=pltpu.CompilerParams(needs_layout_passes=False),
        scratch_types=[
            pltpu.VMEM((CW,), jnp.float32),
            pltpu.VMEM((CW,), jnp.float32),
            pltpu.VMEM((BATCH,), jnp.int32),
            pltpu.VMEM((BATCH,), jnp.float32),
            pltpu.VMEM((BATCH,), jnp.float32),
            pltpu.SemaphoreType.DMA,
        ],
    )
    def k(uid_h, mid_h, gid_h, ut_h, mt_h, gt_h, utl_h, mtl_h, gtl_h,
          ou_h, om_h, og_h, row0, row1, idxb, ob0, ob1, sem):
        wid = lax.axis_index("s") * NUM_CORES + lax.axis_index("c")
        c0 = wid * 2
        iota = lax.iota(jnp.int32, 16)
        for idx_h, tbl_h, tail_h, out_h, ncols in (
            (uid_h, ut_h, utl_h, ou_h, NUM_USERS),
            (mid_h, mt_h, mtl_h, om_h, NUM_MOVIES),
            (gid_h, gt_h, gtl_h, og_h, NUM_GENRES),
        ):
            pltpu.sync_copy(idx_h, idxb)
            tw = ncols % 128
            mcols = ncols - tw
            chunks = [(kk * CW, min(CW, mcols - kk * CW), None)
                      for kk in range(-(-mcols // CW))]
            chunks.append((mcols, tw, tail_h))
            for lo, w, tail in chunks:
                if tail is None:
                    h0 = pltpu.async_copy(tbl_h.at[c0, pl.ds(lo, w)],
                                          row0.at[pl.ds(0, w)], sem)
                    h1 = pltpu.async_copy(tbl_h.at[c0 + 1, pl.ds(lo, w)],
                                          row1.at[pl.ds(0, w)], sem)
                else:
                    h0 = pltpu.async_copy(tail.at[c0],
                                          row0.at[pl.ds(0, 128)], sem)
                    h1 = pltpu.async_copy(tail.at[c0 + 1],
                                          row1.at[pl.ds(0, 128)], sem)
                h0.wait()
                h1.wait()
                hi = lo + w

                def scan(j, carry):
                    pos = j * 16 + iota
                    v = idxb[pl.ds(j * 16, 16)]
                    m = (v >= lo) & (v < hi)
                    loc = jnp.clip(v - lo, 0, w - 1)
                    g0 = plsc.load_gather(row0, [loc], mask=m)
                    plsc.store_scatter(ob0, [pos], g0, mask=m)
                    g1 = plsc.load_gather(row1, [loc], mask=m)
                    plsc.store_scatter(ob1, [pos], g1, mask=m)
                    return carry

                lax.fori_loop(0, NVEC, scan, 0)
            pltpu.sync_copy(ob0, out_h.at[c0])
            pltpu.sync_copy(ob1, out_h.at[c0 + 1])

    return k(uid, mid, gid, utT, mtT, gtT, utail, mtail, gtail)


def _tail(tbl):
    tw = tbl.shape[0] % 128
    return jnp.pad(tbl[tbl.shape[0] - tw:].T, ((0, 0), (0, 128 - tw)))


@jax.jit
def kernel(uid, mid, gid, user_table, movie_table, genre_table):
    ouT, omT, ogT = _gather3(
        uid.astype(jnp.int32), mid.astype(jnp.int32), gid.astype(jnp.int32),
        user_table.T, movie_table.T, genre_table.T,
        _tail(user_table), _tail(movie_table), _tail(genre_table))
    return (ouT.T, omT.T, ogT.T)


# speedup vs baseline: 1.1936x; 1.1936x over previous
import functools

import jax
import jax.numpy as jnp
from jax import lax
from jax.experimental import pallas as pl
from jax.experimental.pallas import tpu as pltpu
from jax.experimental.pallas import tpu_sc as plsc

BATCH = 16384
EMBED_DIM = 64
NUM_USERS = 1000000
NUM_MOVIES = 100000
NUM_GENRES = 1000
NUM_CORES = 2
NUM_SUBCORES = 16
NUM_WORKERS = NUM_CORES * NUM_SUBCORES
CW = 40704
NVEC = BATCH // 16
TABLES = ((NUM_USERS,), (NUM_MOVIES,), (NUM_GENRES,))


def _gather3(uid, mid, gid, utT, mtT, gtT, utail, mtail, gtail):
    mesh = plsc.VectorSubcoreMesh(core_axis_name="c", subcore_axis_name="s")
    out = jax.ShapeDtypeStruct((EMBED_DIM, BATCH), jnp.float32)

    @functools.partial(
        pl.kernel,
        mesh=mesh,
        out_type=(out, out, out),
        compiler_params=pltpu.CompilerParams(needs_layout_passes=False),
        scratch_types=[
            pltpu.VMEM((CW,), jnp.float32),
            pltpu.VMEM((CW,), jnp.float32),
            pltpu.VMEM((BATCH,), jnp.int32),
            pltpu.VMEM((BATCH,), jnp.float32),
            pltpu.VMEM((BATCH,), jnp.float32),
            pltpu.SemaphoreType.DMA,
        ],
    )
    def k(uid_h, mid_h, gid_h, ut_h, mt_h, gt_h, utl_h, mtl_h, gtl_h,
          ou_h, om_h, og_h, row0, row1, idxb, ob0, ob1, sem):
        wid = lax.axis_index("s") * NUM_CORES + lax.axis_index("c")
        c0 = wid * 2
        iota = lax.iota(jnp.int32, 16)
        for idx_h, tbl_h, tail_h, out_h, ncols in (
            (uid_h, ut_h, utl_h, ou_h, NUM_USERS),
            (mid_h, mt_h, mtl_h, om_h, NUM_MOVIES),
            (gid_h, gt_h, gtl_h, og_h, NUM_GENRES),
        ):
            pltpu.sync_copy(idx_h, idxb)
            tw = ncols % 128
            mcols = ncols - tw
            nch = -(-mcols // CW)
            for kk in range(nch):
                lo = kk * CW
                w = min(CW, mcols - lo)
                last = kk == nch - 1
                handles = [
                    pltpu.async_copy(tbl_h.at[c0, pl.ds(lo, w)],
                                     row0.at[pl.ds(0, w)], sem),
                    pltpu.async_copy(tbl_h.at[c0 + 1, pl.ds(lo, w)],
                                     row1.at[pl.ds(0, w)], sem),
                ]
                if last:
                    handles.append(pltpu.async_copy(
                        tail_h.at[c0], row0.at[pl.ds(w, 128)], sem))
                    handles.append(pltpu.async_copy(
                        tail_h.at[c0 + 1], row1.at[pl.ds(w, 128)], sem))
                for h in handles:
                    h.wait()
                we = jnp.uint32(w + tw if last else w)

                def scan(t, carry):
                    posv = t * 128 + iota
                    for u in range(8):
                        off = t * 128 + u * 16
                        v = idxb[pl.ds(off, 16)]
                        d = v - lo
                        m = d.astype(jnp.uint32) < we
                        loc = d & 0xFFFF
                        g0 = plsc.load_gather(row0, [loc], mask=m)
                        plsc.store_scatter(ob0, [posv + u * 16], g0, mask=m)
                        g1 = plsc.load_gather(row1, [loc], mask=m)
                        plsc.store_scatter(ob1, [posv + u * 16], g1, mask=m)
                    return carry

                lax.fori_loop(0, NVEC // 8, scan, 0)
            pltpu.sync_copy(ob0, out_h.at[c0])
            pltpu.sync_copy(ob1, out_h.at[c0 + 1])

    return k(uid, mid, gid, utT, mtT, gtT, utail, mtail, gtail)


def _tail(tbl):
    tw = tbl.shape[0] % 128
    return jnp.pad(tbl[tbl.shape[0] - tw:].T, ((0, 0), (0, 128 - tw)))


@jax.jit
def kernel(uid, mid, gid, user_table, movie_table, genre_table):
    ouT, omT, ogT = _gather3(
        uid.astype(jnp.int32), mid.astype(jnp.int32), gid.astype(jnp.int32),
        user_table.T, movie_table.T, genre_table.T,
        _tail(user_table), _tail(movie_table), _tail(genre_table))
    return (ouT.T, omT.T, ogT.T)


# speedup vs baseline: 2.8261x; 2.3677x over previous
import functools

import jax
import jax.numpy as jnp
from jax import lax
from jax.experimental import pallas as pl
from jax.experimental.pallas import tpu as pltpu
from jax.experimental.pallas import tpu_sc as plsc

BATCH = 16384
EMBED_DIM = 64
NUM_USERS = 1000000
NUM_MOVIES = 100000
NUM_GENRES = 1000
NUM_CORES = 2
NUM_SUBCORES = 16
NUM_WORKERS = NUM_CORES * NUM_SUBCORES
CW = 20224
NVEC = BATCH // 16


def _gather3(uid, mid, gid, utT, mtT, gtT, utail, mtail, gtail):
    mesh = plsc.VectorSubcoreMesh(core_axis_name="c", subcore_axis_name="s")
    out = jax.ShapeDtypeStruct((EMBED_DIM, BATCH), jnp.float32)

    @functools.partial(
        pl.kernel,
        mesh=mesh,
        out_type=(out, out, out),
        compiler_params=pltpu.CompilerParams(needs_layout_passes=False),
        scratch_types=[
            pltpu.VMEM((CW + 128,), jnp.float32),
            pltpu.VMEM((CW + 128,), jnp.float32),
            pltpu.VMEM((CW + 128,), jnp.float32),
            pltpu.VMEM((CW + 128,), jnp.float32),
            pltpu.VMEM((BATCH,), jnp.int32),
            pltpu.VMEM((BATCH,), jnp.float32),
            pltpu.VMEM((BATCH,), jnp.float32),
            pltpu.SemaphoreType.DMA,
            pltpu.SemaphoreType.DMA,
        ],
    )
    def k(uid_h, mid_h, gid_h, ut_h, mt_h, gt_h, utl_h, mtl_h, gtl_h,
          ou_h, om_h, og_h, ra0, ra1, rb0, rb1, idxb, ob0, ob1, s_a, s_b):
        wid = lax.axis_index("s") * NUM_CORES + lax.axis_index("c")
        c0 = wid * 2
        iota = lax.iota(jnp.int32, 16)
        rows = ((ra0, ra1), (rb0, rb1))
        sems = (s_a, s_b)
        for idx_h, tbl_h, tail_h, out_h, ncols in (
            (uid_h, ut_h, utl_h, ou_h, NUM_USERS),
            (mid_h, mt_h, mtl_h, om_h, NUM_MOVIES),
            (gid_h, gt_h, gtl_h, og_h, NUM_GENRES),
        ):
            pltpu.sync_copy(idx_h, idxb)
            tw = ncols % 128
            mcols = ncols - tw
            nfull = mcols // CW
            rem = mcols - nfull * CW
            elo = nfull * CW
            ewe = jnp.uint32(rem + tw)

            def scan_chunk(lo, we, row0, row1):
                def scan(t, carry):
                    posv = t * 128 + iota
                    vs = [idxb[pl.ds(t * 128 + u * 16, 16)] for u in range(8)]
                    dd = [v - lo for v in vs]
                    ms = [d.astype(jnp.uint32) < we for d in dd]
                    locs = [d & 0xFFFF for d in dd]
                    g0s = [plsc.load_gather(row0, [loc], mask=m)
                           for loc, m in zip(locs, ms)]
                    g1s = [plsc.load_gather(row1, [loc], mask=m)
                           for loc, m in zip(locs, ms)]
                    for u in range(8):
                        plsc.store_scatter(ob0, [posv + u * 16], g0s[u],
                                           mask=ms[u])
                        plsc.store_scatter(ob1, [posv + u * 16], g1s[u],
                                           mask=ms[u])
                    return carry

                lax.fori_loop(0, NVEC // 8, scan, 0)

            def fire_full(i, par):
                lo = pl.multiple_of(i * CW, CW)
                r0, r1 = rows[par]
                sm = sems[par]
                pltpu.async_copy(tbl_h.at[c0, pl.ds(lo, CW)],
                                 r0.at[pl.ds(0, CW)], sm)
                pltpu.async_copy(tbl_h.at[c0 + 1, pl.ds(lo, CW)],
                                 r1.at[pl.ds(0, CW)], sm)

            def fire_epi():
                r0, r1 = rows[nfull % 2]
                sm = sems[nfull % 2]
                if rem:
                    pltpu.async_copy(tbl_h.at[c0, pl.ds(elo, rem)],
                                     r0.at[pl.ds(0, rem)], sm)
                    pltpu.async_copy(tbl_h.at[c0 + 1, pl.ds(elo, rem)],
                                     r1.at[pl.ds(0, rem)], sm)
                pltpu.async_copy(tail_h.at[c0], r0.at[pl.ds(rem, 128)], sm)
                pltpu.async_copy(tail_h.at[c0 + 1], r1.at[pl.ds(rem, 128)],
                                 sm)

            def wait_epi():
                r0, r1 = rows[nfull % 2]
                sm = sems[nfull % 2]
                if rem:
                    pltpu.make_async_copy(tbl_h.at[c0, pl.ds(0, rem)],
                                          r0.at[pl.ds(0, rem)], sm).wait()
                    pltpu.make_async_copy(tbl_h.at[c0, pl.ds(0, rem)],
                                          r1.at[pl.ds(0, rem)], sm).wait()
                pltpu.make_async_copy(tail_h.at[c0],
                                      r0.at[pl.ds(rem, 128)], sm).wait()
                pltpu.make_async_copy(tail_h.at[c0],
                                      r1.at[pl.ds(rem, 128)], sm).wait()

            if nfull:
                fire_full(0, 0)

                def body(i, par):
                    @pl.when(i + 1 < nfull)
                    def _():
                        fire_full(i + 1, 1 - par)

                    if (nfull % 2) == (1 - par):
                        @pl.when(i + 1 == nfull)
                        def _():
                            fire_epi()

                    r0, r1 = rows[par]
                    sm = sems[par]
                    pltpu.make_async_copy(tbl_h.at[c0, pl.ds(0, CW)],
                                          r0.at[pl.ds(0, CW)], sm).wait()
                    pltpu.make_async_copy(tbl_h.at[c0, pl.ds(0, CW)],
                                          r1.at[pl.ds(0, CW)], sm).wait()
                    scan_chunk(i * CW, jnp.uint32(CW), r0, r1)

                def body2(p, carry):
                    i0 = p * 2
                    body(i0, 0)

                    @pl.when(i0 + 1 < nfull)
                    def _():
                        body(i0 + 1, 1)

                    return carry

                lax.fori_loop(0, -(-nfull // 2), body2, 0)
            else:
                fire_epi()
            wait_epi()
            r0, r1 = rows[nfull % 2]
            scan_chunk(elo, ewe, r0, r1)
            pltpu.sync_copy(ob0, out_h.at[c0])
            pltpu.sync_copy(ob1, out_h.at[c0 + 1])

    return k(uid, mid, gid, utT, mtT, gtT, utail, mtail, gtail)


def _tail(tbl):
    tw = tbl.shape[0] % 128
    return jnp.pad(tbl[tbl.shape[0] - tw:].T, ((0, 0), (0, 128 - tw)))


@jax.jit
def kernel(uid, mid, gid, user_table, movie_table, genre_table):
    ouT, omT, ogT = _gather3(
        uid.astype(jnp.int32), mid.astype(jnp.int32), gid.astype(jnp.int32),
        user_table.T, movie_table.T, genre_table.T,
        _tail(user_table), _tail(movie_table), _tail(genre_table))
    return (ouT.T, omT.T, ogT.T)
